# Initial kernel scaffold; baseline (speedup 1.0000x reference)
import jax
import jax.numpy as jnp
from jax.experimental import pallas as pl


def kernel(input, target):
    raise NotImplementedError("write your pallas kernel here")


# speedup vs baseline: 24.4163x; 24.4163x over previous
import functools

import jax
import jax.numpy as jnp
from jax import lax
from jax.experimental import pallas as pl
from jax.experimental.pallas import tpu as pltpu
from jax.experimental.pallas import tpu_sc as plsc

_RATIO = 0.1
_NW = 32
_NB = 2048
_NBC = _NB // 16


def _diff_body(a_ref, b_ref, o_ref):
    o_ref[...] = jnp.abs(a_ref[...] - b_ref[...]).astype(jnp.bfloat16)


def _diff(x, y):
    rows, hw = x.shape
    br, bhw = 32, 18432
    return pl.pallas_call(
        _diff_body,
        out_shape=jax.ShapeDtypeStruct((rows, hw), jnp.bfloat16),
        grid=(rows // br, hw // bhw),
        in_specs=[
            pl.BlockSpec((br, bhw), lambda i, j: (i, j)),
            pl.BlockSpec((br, bhw), lambda i, j: (i, j)),
        ],
        out_specs=pl.BlockSpec((br, bhw), lambda i, j: (i, j)),
    )(x, y)


def _mean_body(p_ref, o_ref, *, denom):
    o_ref[...] = jnp.reshape(jnp.sum(p_ref[...]) * (1.0 / denom), (1, 1))


def _mean(p, denom):
    return pl.pallas_call(
        functools.partial(_mean_body, denom=denom),
        out_shape=jax.ShapeDtypeStruct((1, 1), jnp.float32),
    )(p)


def _make_sc_topk(rows, hw, k):
    rpw = rows // _NW
    npairs = hw // 32
    mesh = plsc.VectorSubcoreMesh(core_axis_name="c", subcore_axis_name="s")

    @functools.partial(
        pl.kernel,
        out_type=jax.ShapeDtypeStruct((_NW, 128), jnp.float32),
        mesh=mesh,
        compiler_params=pltpu.CompilerParams(
            use_tc_tiling_on_sc=False, needs_layout_passes=False),
        scratch_types=[
            pltpu.VMEM((hw,), jnp.bfloat16),
            pltpu.VMEM((16 * _NB,), jnp.int32),
            pltpu.VMEM((_NB,), jnp.int32),
            pltpu.VMEM((256,), jnp.int32),
            pltpu.VMEM((128,), jnp.float32),
            pltpu.SemaphoreType.DMA,
        ],
    )
    def sc_topk(diff_hbm, out_hbm, row_buf, hist, counts, hist2, obuf, sem):
        cid = lax.axis_index("c")
        sid = lax.axis_index("s")
        wid = sid * 2 + cid
        lanes = lax.iota(jnp.int32, 16)
        ones = jnp.ones((16,), jnp.int32)
        izeros = jnp.zeros((16,), jnp.int32)
        fzeros = jnp.zeros((16,), jnp.float32)
        lane_hist = lanes * _NB
        lane_h2 = lanes * 16

        def row_step(i, acc):
            r = wid * rpw + i
            pltpu.sync_copy(diff_hbm.at[r], row_buf)

            def zero_hist(j, c):
                hist[pl.ds(j * 16, 16)] = izeros
                return c
            lax.fori_loop(0, 16 * _NB // 16, zero_hist, 0)

            def p1(j, c):
                va, vb = plsc.unpack(row_buf[pl.ds(j * 32, 32)],
                                     format=plsc.PackFormat.INTERLEAVED)
                for v in (va, vb):
                    b = lax.shift_right_logical(
                        plsc.bitcast(v, jnp.int32), 20)
                    plsc.addupdate_scatter(hist, [lane_hist + b], ones)
                return c
            lax.fori_loop(0, npairs, p1, 0)

            def lred(c, s):
                b0 = c * 16
                v = hist[pl.ds(b0, 16)]
                for l in range(1, 16):
                    v = v + hist[pl.ds(l * _NB + b0, 16)]
                counts[pl.ds(b0, 16)] = v
                return s
            lax.fori_loop(0, _NBC, lred, 0)

            def walk(c, st):
                carry, found, tb, need = st
                b0 = (_NBC - 1 - c) * 16
                v = counts[pl.ds(b0, 16)]
                rev = lax.rev(v, (0,))
                cs = plsc.cumsum(rev) + carry
                reached = cs >= k
                nreach = jnp.sum(reached.astype(jnp.int32))
                p = jnp.min(jnp.where(reached, lanes, 16))
                cumbefore = cs - rev
                cb_p = jnp.sum(jnp.where(lanes == p, cumbefore, 0))
                upd = jnp.logical_and(nreach > 0, found == 0)
                tb = jnp.where(upd, b0 + 15 - p, tb)
                need = jnp.where(upd, k - cb_p, need)
                found = jnp.where(upd, 1, found)
                return (carry + jnp.sum(v), found, tb, need)
            _, _, tb, need = lax.fori_loop(
                0, _NBC, walk,
                (jnp.int32(0), jnp.int32(0), jnp.int32(0), jnp.int32(0)))

            for l in range(16):
                hist2[pl.ds(l * 16, 16)] = izeros

            def p2(j, accv):
                va, vb = plsc.unpack(row_buf[pl.ds(j * 32, 32)],
                                     format=plsc.PackFormat.INTERLEAVED)
                for v in (va, vb):
                    bits = plsc.bitcast(v, jnp.int32)
                    b = lax.shift_right_logical(bits, 20)
                    accv = accv + jnp.where(b > tb, v, 0.0)
                    plsc.addupdate_scatter(
                        hist2,
                        [lane_h2 + (lax.shift_right_logical(bits, 16) & 15)],
                        ones, mask=b == tb)
                return accv
            accv = lax.fori_loop(0, npairs, p2, fzeros)

            c2 = hist2[pl.ds(0, 16)]
            for l in range(1, 16):
                c2 = c2 + hist2[pl.ds(l * 16, 16)]
            rev2 = lax.rev(c2, (0,))
            cs2 = plsc.cumsum(rev2)
            take = jnp.clip(need - (cs2 - rev2), 0, rev2)
            pat = lax.shift_left(tb, 4) + (15 - lanes)
            vals = plsc.bitcast(lax.shift_left(pat, 16), jnp.float32)
            return acc + accv + take.astype(jnp.float32) * vals

        acc = lax.fori_loop(0, rpw, row_step, fzeros)
        for l in range(8):
            obuf[pl.ds(l * 16, 16)] = fzeros
        obuf[pl.ds(0, 16)] = acc
        pltpu.sync_copy(obuf, out_hbm.at[wid])

    return sc_topk


def kernel(input, target):
    n, c, h, w = input.shape
    rows, hw = n * c, h * w
    k = int(hw * _RATIO)
    x = input.reshape(rows, hw)
    y = target.reshape(rows, hw)
    diff16 = _diff(x, y)
    partials = _make_sc_topk(rows, hw, k)(diff16)
    out = _mean(partials, float(rows) * float(k))
    return out.reshape(())


# speedup vs baseline: 32.0846x; 1.3141x over previous
import functools

import jax
import jax.numpy as jnp
from jax import lax
from jax.experimental import pallas as pl
from jax.experimental.pallas import tpu as pltpu
from jax.experimental.pallas import tpu_sc as plsc

_RATIO = 0.1
_NW = 32
_NB = 2048
_NBC = _NB // 16


def _pat16(a, b):
    bits = jax.lax.bitcast_convert_type(jnp.abs(a - b), jnp.int32)
    return lax.shift_right_logical(
        bits + 0x7FFF + (lax.shift_right_logical(bits, 16) & 1), 16)


def _diff_body(alo_ref, ahi_ref, blo_ref, bhi_ref, o_ref):
    plo = _pat16(alo_ref[...], blo_ref[...])
    phi = _pat16(ahi_ref[...], bhi_ref[...])
    o_ref[...] = plo | lax.shift_left(phi, 16)


def _diff(x, y):
    rows, hw = x.shape
    half = hw // 2
    br, bc = 32, 9216
    ncb = half // bc
    lo = lambda i, j: (i, j)
    hi = lambda i, j: (i, j + ncb)
    return pl.pallas_call(
        _diff_body,
        out_shape=jax.ShapeDtypeStruct((rows, half), jnp.int32),
        grid=(rows // br, ncb),
        in_specs=[
            pl.BlockSpec((br, bc), lo),
            pl.BlockSpec((br, bc), hi),
            pl.BlockSpec((br, bc), lo),
            pl.BlockSpec((br, bc), hi),
        ],
        out_specs=pl.BlockSpec((br, bc), lo),
    )(x, x, y, y)


def _mean_body(p_ref, o_ref, *, denom):
    o_ref[...] = jnp.reshape(jnp.sum(p_ref[...]) * (1.0 / denom), (1, 1))


def _mean(p, denom):
    return pl.pallas_call(
        functools.partial(_mean_body, denom=denom),
        out_shape=jax.ShapeDtypeStruct((1, 1), jnp.float32),
    )(p)


def _make_sc_topk(rows, hw, k):
    rpw = rows // _NW
    nwords = hw // 32
    mesh = plsc.VectorSubcoreMesh(core_axis_name="c", subcore_axis_name="s")

    @functools.partial(
        pl.kernel,
        out_type=jax.ShapeDtypeStruct((_NW, 128), jnp.float32),
        mesh=mesh,
        compiler_params=pltpu.CompilerParams(
            use_tc_tiling_on_sc=False, needs_layout_passes=False),
        scratch_types=[
            pltpu.VMEM((hw // 2,), jnp.int32),
            pltpu.VMEM((16 * _NB,), jnp.int32),
            pltpu.VMEM((_NB,), jnp.int32),
            pltpu.VMEM((256,), jnp.int32),
            pltpu.VMEM((128,), jnp.float32),
            pltpu.SemaphoreType.DMA,
        ],
    )
    def sc_topk(diff_hbm, out_hbm, row_buf, hist, counts, hist2, obuf, sem):
        cid = lax.axis_index("c")
        sid = lax.axis_index("s")
        wid = sid * 2 + cid
        lanes = lax.iota(jnp.int32, 16)
        ones = jnp.ones((16,), jnp.int32)
        izeros = jnp.zeros((16,), jnp.int32)
        fzeros = jnp.zeros((16,), jnp.float32)
        lane_hist = lanes * _NB
        lane_h2 = lanes * 16

        def zero_hist(j, c):
            hist[pl.ds(j * 16, 16)] = izeros
            return c
        lax.fori_loop(0, 16 * _NB // 16, zero_hist, 0)

        def row_step(i, acc):
            r = wid * rpw + i
            pltpu.sync_copy(diff_hbm.at[r], row_buf)

            def p1(j, c):
                for t in range(4):
                    u = row_buf[pl.ds((j * 4 + t) * 16, 16)]
                    blo = lax.shift_right_logical(u & 0xFFFF, 4)
                    bhi = lax.shift_right_logical(u, 20)
                    plsc.addupdate_scatter(hist, [lane_hist + blo], ones)
                    plsc.addupdate_scatter(hist, [lane_hist + bhi], ones)
                return c
            lax.fori_loop(0, nwords // 4, p1, 0)

            def lred(c, s):
                b0 = c * 16
                v = hist[pl.ds(b0, 16)]
                hist[pl.ds(b0, 16)] = izeros
                for l in range(1, 16):
                    o = l * _NB + b0
                    v = v + hist[pl.ds(o, 16)]
                    hist[pl.ds(o, 16)] = izeros
                counts[pl.ds(b0, 16)] = v
                return s
            lax.fori_loop(0, _NBC, lred, 0)

            def walk(c, st):
                carry, found, tb, need = st
                b0 = (_NBC - 1 - c) * 16
                v = counts[pl.ds(b0, 16)]
                rev = lax.rev(v, (0,))
                cs = plsc.cumsum(rev) + carry
                reached = cs >= k
                nreach = jnp.sum(reached.astype(jnp.int32))
                p = jnp.min(jnp.where(reached, lanes, 16))
                cumbefore = cs - rev
                cb_p = jnp.sum(jnp.where(lanes == p, cumbefore, 0))
                upd = jnp.logical_and(nreach > 0, found == 0)
                tb = jnp.where(upd, b0 + 15 - p, tb)
                need = jnp.where(upd, k - cb_p, need)
                found = jnp.where(upd, 1, found)
                return (carry + jnp.sum(v), found, tb, need)
            _, _, tb, need = lax.fori_loop(
                0, _NBC, walk,
                (jnp.int32(0), jnp.int32(0), jnp.int32(0), jnp.int32(0)))

            for l in range(16):
                hist2[pl.ds(l * 16, 16)] = izeros

            def p2(j, accv):
                for t in range(2):
                    u = row_buf[pl.ds((j * 2 + t) * 16, 16)]
                    for pat in (u & 0xFFFF, lax.shift_right_logical(u, 16)):
                        b = lax.shift_right_logical(pat, 4)
                        val = plsc.bitcast(lax.shift_left(pat, 16),
                                           jnp.float32)
                        accv = accv + jnp.where(b > tb, val, 0.0)
                        plsc.addupdate_scatter(
                            hist2, [lane_h2 + (pat & 15)], ones,
                            mask=b == tb)
                return accv
            accv = lax.fori_loop(0, nwords // 2, p2, fzeros)

            c2 = hist2[pl.ds(0, 16)]
            for l in range(1, 16):
                c2 = c2 + hist2[pl.ds(l * 16, 16)]
            rev2 = lax.rev(c2, (0,))
            cs2 = plsc.cumsum(rev2)
            take = jnp.clip(need - (cs2 - rev2), 0, rev2)
            pat = lax.shift_left(tb, 4) + (15 - lanes)
            vals = plsc.bitcast(lax.shift_left(pat, 16), jnp.float32)
            return acc + accv + take.astype(jnp.float32) * vals

        acc = lax.fori_loop(0, rpw, row_step, fzeros)
        for l in range(8):
            obuf[pl.ds(l * 16, 16)] = fzeros
        obuf[pl.ds(0, 16)] = acc
        pltpu.sync_copy(obuf, out_hbm.at[wid])

    return sc_topk


def kernel(input, target):
    n, c, h, w = input.shape
    rows, hw = n * c, h * w
    k = int(hw * _RATIO)
    x = input.reshape(rows, hw)
    y = target.reshape(rows, hw)
    packed = _diff(x, y)
    partials = _make_sc_topk(rows, hw, k)(packed)
    out = _mean(partials, float(rows) * float(k))
    return out.reshape(())


# speedup vs baseline: 32.3354x; 1.0078x over previous
import functools

import jax
import jax.numpy as jnp
from jax import lax
from jax.experimental import pallas as pl
from jax.experimental.pallas import tpu as pltpu
from jax.experimental.pallas import tpu_sc as plsc

_RATIO = 0.1
_NW = 32
_NB = 2048
_NBC = _NB // 16


def _pat16(a, b):
    bits = jax.lax.bitcast_convert_type(jnp.abs(a - b), jnp.int32)
    return lax.shift_right_logical(
        bits + 0x7FFF + (lax.shift_right_logical(bits, 16) & 1), 16)


def _diff_body(alo_ref, ahi_ref, blo_ref, bhi_ref, o_ref):
    plo = _pat16(alo_ref[...], blo_ref[...])
    phi = _pat16(ahi_ref[...], bhi_ref[...])
    o_ref[...] = plo | lax.shift_left(phi, 16)


def _diff(x, y):
    rows, hw = x.shape
    half = hw // 2
    br, bc = 32, 9216
    ncb = half // bc
    lo = lambda i, j: (i, j)
    hi = lambda i, j: (i, j + ncb)
    return pl.pallas_call(
        _diff_body,
        out_shape=jax.ShapeDtypeStruct((rows, half), jnp.int32),
        grid=(rows // br, ncb),
        in_specs=[
            pl.BlockSpec((br, bc), lo),
            pl.BlockSpec((br, bc), hi),
            pl.BlockSpec((br, bc), lo),
            pl.BlockSpec((br, bc), hi),
        ],
        out_specs=pl.BlockSpec((br, bc), lo),
    )(x, x, y, y)


def _mean_body(p_ref, o_ref, *, denom):
    o_ref[...] = jnp.reshape(jnp.sum(p_ref[...]) * (1.0 / denom), (1, 1))


def _mean(p, denom):
    return pl.pallas_call(
        functools.partial(_mean_body, denom=denom),
        out_shape=jax.ShapeDtypeStruct((1, 1), jnp.float32),
    )(p)


def _make_sc_topk(rows, hw, k):
    rpw = rows // _NW
    nwords = hw // 32
    mesh = plsc.VectorSubcoreMesh(core_axis_name="c", subcore_axis_name="s")

    @functools.partial(
        pl.kernel,
        out_type=jax.ShapeDtypeStruct((_NW, 128), jnp.float32),
        mesh=mesh,
        compiler_params=pltpu.CompilerParams(needs_layout_passes=False),
        scratch_types=[
            pltpu.VMEM((hw // 256, 128), jnp.int32),
            pltpu.VMEM((16 * _NB,), jnp.int32),
            pltpu.VMEM((_NB,), jnp.int32),
            pltpu.VMEM((256,), jnp.int32),
            pltpu.VMEM((128,), jnp.float32),
            pltpu.SemaphoreType.DMA,
        ],
    )
    def sc_topk(diff_hbm, out_hbm, row_buf, hist, counts, hist2, obuf, sem):
        cid = lax.axis_index("c")
        sid = lax.axis_index("s")
        wid = sid * 2 + cid
        lanes = lax.iota(jnp.int32, 16)
        ones = jnp.ones((16,), jnp.int32)
        izeros = jnp.zeros((16,), jnp.int32)
        fzeros = jnp.zeros((16,), jnp.float32)
        lane_hist = lanes * _NB
        lane_h2 = lanes * 16

        def zero_hist(j, c):
            hist[pl.ds(j * 16, 16)] = izeros
            return c
        lax.fori_loop(0, 16 * _NB // 16, zero_hist, 0)

        def row_step(i, acc):
            r = wid * rpw + i
            pltpu.sync_copy(diff_hbm.at[r], row_buf)

            def p1(s, c):
                for t in range(8):
                    u = row_buf[s, pl.ds(t * 16, 16)]
                    blo = lax.shift_right_logical(u & 0xFFFF, 4)
                    bhi = lax.shift_right_logical(u, 20)
                    plsc.addupdate_scatter(hist, [lane_hist + blo], ones)
                    plsc.addupdate_scatter(hist, [lane_hist + bhi], ones)
                return c
            lax.fori_loop(0, nwords // 8, p1, 0)

            def lred(c, s):
                b0 = c * 16
                v = hist[pl.ds(b0, 16)]
                hist[pl.ds(b0, 16)] = izeros
                for l in range(1, 16):
                    o = l * _NB + b0
                    v = v + hist[pl.ds(o, 16)]
                    hist[pl.ds(o, 16)] = izeros
                counts[pl.ds(b0, 16)] = v
                return s
            lax.fori_loop(0, _NBC, lred, 0)

            def walk(c, st):
                carry, found, tb, need = st
                b0 = (_NBC - 1 - c) * 16
                v = counts[pl.ds(b0, 16)]
                rev = lax.rev(v, (0,))
                cs = plsc.cumsum(rev) + carry
                reached = cs >= k
                nreach = jnp.sum(reached.astype(jnp.int32))
                p = jnp.min(jnp.where(reached, lanes, 16))
                cumbefore = cs - rev
                cb_p = jnp.sum(jnp.where(lanes == p, cumbefore, 0))
                upd = jnp.logical_and(nreach > 0, found == 0)
                tb = jnp.where(upd, b0 + 15 - p, tb)
                need = jnp.where(upd, k - cb_p, need)
                found = jnp.where(upd, 1, found)
                return (carry + jnp.sum(v), found, tb, need)
            _, _, tb, need = lax.fori_loop(
                0, _NBC, walk,
                (jnp.int32(0), jnp.int32(0), jnp.int32(0), jnp.int32(0)))

            for l in range(16):
                hist2[pl.ds(l * 16, 16)] = izeros

            def p2(s, accv):
                for t in range(8):
                    u = row_buf[s, pl.ds(t * 16, 16)]
                    for pat in (u & 0xFFFF, lax.shift_right_logical(u, 16)):
                        b = lax.shift_right_logical(pat, 4)
                        val = plsc.bitcast(lax.shift_left(pat, 16),
                                           jnp.float32)
                        accv = accv + jnp.where(b > tb, val, 0.0)
                        plsc.addupdate_scatter(
                            hist2, [lane_h2 + (pat & 15)], ones,
                            mask=b == tb)
                return accv
            accv = lax.fori_loop(0, nwords // 8, p2, fzeros)

            c2 = hist2[pl.ds(0, 16)]
            for l in range(1, 16):
                c2 = c2 + hist2[pl.ds(l * 16, 16)]
            rev2 = lax.rev(c2, (0,))
            cs2 = plsc.cumsum(rev2)
            take = jnp.clip(need - (cs2 - rev2), 0, rev2)
            pat = lax.shift_left(tb, 4) + (15 - lanes)
            vals = plsc.bitcast(lax.shift_left(pat, 16), jnp.float32)
            return acc + accv + take.astype(jnp.float32) * vals

        acc = lax.fori_loop(0, rpw, row_step, fzeros)
        for l in range(8):
            obuf[pl.ds(l * 16, 16)] = fzeros
        obuf[pl.ds(0, 16)] = acc
        pltpu.sync_copy(obuf, out_hbm.at[wid])

    return sc_topk


def kernel(input, target):
    n, c, h, w = input.shape
    rows, hw = n * c, h * w
    k = int(hw * _RATIO)
    x = input.reshape(rows, hw)
    y = target.reshape(rows, hw)
    packed = _diff(x, y).reshape(rows, hw // 256, 128)
    partials = _make_sc_topk(rows, hw, k)(packed)
    out = _mean(partials, float(rows) * float(k))
    return out.reshape(())


# speedup vs baseline: 53.4847x; 1.6541x over previous
import functools

import jax
import jax.numpy as jnp
from jax import lax
from jax.experimental import pallas as pl
from jax.experimental.pallas import tpu as pltpu
from jax.experimental.pallas import tpu_sc as plsc

_RATIO = 0.1
_NW = 32
_NB = 2048
_NBC = _NB // 16


def _pat16(a, b):
    bits = jax.lax.bitcast_convert_type(jnp.abs(a - b), jnp.int32)
    return lax.shift_right_logical(
        bits + 0x7FFF + (lax.shift_right_logical(bits, 16) & 1), 16)


def _diff_body(alo_ref, ahi_ref, blo_ref, bhi_ref, o_ref):
    plo = _pat16(alo_ref[...], blo_ref[...])
    phi = _pat16(ahi_ref[...], bhi_ref[...])
    o_ref[...] = plo | lax.shift_left(phi, 16)


def _diff(x, y):
    rows, hw = x.shape
    half = hw // 2
    br, bc = 32, 9216
    ncb = half // bc
    lo = lambda i, j: (i, j)
    hi = lambda i, j: (i, j + ncb)
    return pl.pallas_call(
        _diff_body,
        out_shape=jax.ShapeDtypeStruct((rows, half), jnp.int32),
        grid=(rows // br, ncb),
        in_specs=[
            pl.BlockSpec((br, bc), lo),
            pl.BlockSpec((br, bc), hi),
            pl.BlockSpec((br, bc), lo),
            pl.BlockSpec((br, bc), hi),
        ],
        out_specs=pl.BlockSpec((br, bc), lo),
    )(x, x, y, y)


def _mean_body(p_ref, o_ref, *, denom):
    o_ref[...] = jnp.reshape(jnp.sum(p_ref[...]) * (1.0 / denom), (1, 1))


def _mean(p, denom):
    return pl.pallas_call(
        functools.partial(_mean_body, denom=denom),
        out_shape=jax.ShapeDtypeStruct((1, 1), jnp.float32),
    )(p)


def _make_sc_topk(rows, hw, k):
    rpw = rows // _NW
    nwords = hw // 32
    mesh = plsc.VectorSubcoreMesh(core_axis_name="c", subcore_axis_name="s")

    @functools.partial(
        pl.kernel,
        out_type=jax.ShapeDtypeStruct((_NW, 128), jnp.float32),
        mesh=mesh,
        compiler_params=pltpu.CompilerParams(needs_layout_passes=False),
        scratch_types=[
            pltpu.VMEM((hw // 256, 128), jnp.int32),
            pltpu.VMEM((16 * _NB,), jnp.int32),
            pltpu.VMEM((_NB,), jnp.int32),
            pltpu.VMEM((256,), jnp.int32),
            pltpu.VMEM((128,), jnp.float32),
            pltpu.SemaphoreType.DMA,
        ],
    )
    def sc_topk(diff_hbm, out_hbm, row_buf, hist, counts, hist2, obuf, sem):
        cid = lax.axis_index("c")
        sid = lax.axis_index("s")
        wid = sid * 2 + cid
        lanes = lax.iota(jnp.int32, 16)
        ones = jnp.ones((16,), jnp.int32)
        izeros = jnp.zeros((16,), jnp.int32)
        fzeros = jnp.zeros((16,), jnp.float32)
        lane_hist = lanes * _NB
        lane_h2 = lanes * 16

        @plsc.parallel_loop(0, 16 * _NB // 16, unroll=4)
        def zero_hist(j):
            hist[pl.ds(j * 16, 16)] = izeros

        def row_step(i, acc):
            r = wid * rpw + i
            pltpu.sync_copy(diff_hbm.at[r], row_buf)

            @plsc.parallel_loop(0, nwords // 8, unroll=2)
            def p1(s):
                for t in range(8):
                    u = row_buf[s, pl.ds(t * 16, 16)]
                    blo = lax.shift_right_logical(u & 0xFFFF, 4)
                    bhi = lax.shift_right_logical(u, 20)
                    plsc.addupdate_scatter(hist, [lane_hist + blo], ones)
                    plsc.addupdate_scatter(hist, [lane_hist + bhi], ones)

            @plsc.parallel_loop(0, _NBC, unroll=2)
            def lred(c):
                b0 = c * 16
                v = hist[pl.ds(b0, 16)]
                hist[pl.ds(b0, 16)] = izeros
                for l in range(1, 16):
                    o = l * _NB + b0
                    v = v + hist[pl.ds(o, 16)]
                    hist[pl.ds(o, 16)] = izeros
                counts[pl.ds(b0, 16)] = v

            def walk(c, st):
                carry, found, tb, need = st
                b0 = (_NBC - 1 - c) * 16
                v = counts[pl.ds(b0, 16)]
                rev = lax.rev(v, (0,))
                cs = plsc.cumsum(rev) + carry
                reached = cs >= k
                nreach = jnp.sum(reached.astype(jnp.int32))
                p = jnp.min(jnp.where(reached, lanes, 16))
                cumbefore = cs - rev
                cb_p = jnp.sum(jnp.where(lanes == p, cumbefore, 0))
                upd = jnp.logical_and(nreach > 0, found == 0)
                tb = jnp.where(upd, b0 + 15 - p, tb)
                need = jnp.where(upd, k - cb_p, need)
                found = jnp.where(upd, 1, found)
                return (carry + jnp.sum(v), found, tb, need)
            _, _, tb, need = lax.fori_loop(
                0, _NBC, walk,
                (jnp.int32(0), jnp.int32(0), jnp.int32(0), jnp.int32(0)))

            for l in range(16):
                hist2[pl.ds(l * 16, 16)] = izeros

            @plsc.parallel_loop(0, nwords // 8, unroll=2,
                                carry=(fzeros, fzeros, fzeros, fzeros))
            def p2(s, accs):
                accs = list(accs)
                idx = 0
                for t in range(8):
                    u = row_buf[s, pl.ds(t * 16, 16)]
                    for pat in (u & 0xFFFF, lax.shift_right_logical(u, 16)):
                        b = lax.shift_right_logical(pat, 4)
                        val = plsc.bitcast(lax.shift_left(pat, 16),
                                           jnp.float32)
                        accs[idx] = accs[idx] + jnp.where(b > tb, val, 0.0)
                        idx = (idx + 1) % 4
                        plsc.addupdate_scatter(
                            hist2, [lane_h2 + (pat & 15)], ones,
                            mask=b == tb)
                return tuple(accs)
            accv = p2[0] + p2[1] + p2[2] + p2[3]

            c2 = hist2[pl.ds(0, 16)]
            for l in range(1, 16):
                c2 = c2 + hist2[pl.ds(l * 16, 16)]
            rev2 = lax.rev(c2, (0,))
            cs2 = plsc.cumsum(rev2)
            take = jnp.clip(need - (cs2 - rev2), 0, rev2)
            pat = lax.shift_left(tb, 4) + (15 - lanes)
            vals = plsc.bitcast(lax.shift_left(pat, 16), jnp.float32)
            return acc + accv + take.astype(jnp.float32) * vals

        acc = lax.fori_loop(0, rpw, row_step, fzeros)
        for l in range(8):
            obuf[pl.ds(l * 16, 16)] = fzeros
        obuf[pl.ds(0, 16)] = acc
        pltpu.sync_copy(obuf, out_hbm.at[wid])

    return sc_topk


def kernel(input, target):
    n, c, h, w = input.shape
    rows, hw = n * c, h * w
    k = int(hw * _RATIO)
    x = input.reshape(rows, hw)
    y = target.reshape(rows, hw)
    packed = _diff(x, y).reshape(rows, hw // 256, 128)
    partials = _make_sc_topk(rows, hw, k)(packed)
    out = _mean(partials, float(rows) * float(k))
    return out.reshape(())


# speedup vs baseline: 76.6939x; 1.4339x over previous
import functools

import jax
import jax.numpy as jnp
from jax import lax
from jax.experimental import pallas as pl
from jax.experimental.pallas import tpu as pltpu
from jax.experimental.pallas import tpu_sc as plsc

_RATIO = 0.1
_NW = 32
_NB = 2048
_NBC = _NB // 16


def _pat16(a, b):
    bits = jax.lax.bitcast_convert_type(jnp.abs(a - b), jnp.int32)
    return lax.shift_right_logical(
        bits + 0x7FFF + (lax.shift_right_logical(bits, 16) & 1), 16)


def _diff_body(a_ref, b_ref, o_ref, *, hh):
    a = a_ref[...]
    b = b_ref[...]
    plo = _pat16(a[:, :, :hh, :], b[:, :, :hh, :])
    phi = _pat16(a[:, :, hh:, :], b[:, :, hh:, :])
    o_ref[...] = plo | lax.shift_left(phi, 16)


def _diff(x, y):
    n, c, h, w = x.shape
    hh = h // 2
    bc = 8
    spec = pl.BlockSpec((1, bc, h, w), lambda i, j: (i, j, 0, 0))
    return pl.pallas_call(
        functools.partial(_diff_body, hh=hh),
        out_shape=jax.ShapeDtypeStruct((n, c, hh, w), jnp.int32),
        grid=(n, c // bc),
        in_specs=[spec, spec],
        out_specs=pl.BlockSpec((1, bc, hh, w), lambda i, j: (i, j, 0, 0)),
    )(x, y)


def _mean_body(p_ref, o_ref, *, denom):
    o_ref[...] = jnp.reshape(jnp.sum(p_ref[...]) * (1.0 / denom), (1, 1))


def _mean(p, denom):
    return pl.pallas_call(
        functools.partial(_mean_body, denom=denom),
        out_shape=jax.ShapeDtypeStruct((1, 1), jnp.float32),
    )(p)


def _make_sc_topk(n, c, hh, w, k):
    rows = n * c
    rpw = rows // _NW
    ngrp = w // 16
    mesh = plsc.VectorSubcoreMesh(core_axis_name="c", subcore_axis_name="s")

    @functools.partial(
        pl.kernel,
        out_type=jax.ShapeDtypeStruct((_NW, 128), jnp.float32),
        mesh=mesh,
        compiler_params=pltpu.CompilerParams(needs_layout_passes=False),
        scratch_types=[
            pltpu.VMEM((hh, w), jnp.int32),
            pltpu.VMEM((16 * _NB,), jnp.int32),
            pltpu.VMEM((_NB,), jnp.int32),
            pltpu.VMEM((256,), jnp.int32),
            pltpu.VMEM((128,), jnp.float32),
            pltpu.SemaphoreType.DMA,
        ],
    )
    def sc_topk(diff_hbm, out_hbm, row_buf, hist, counts, hist2, obuf, sem):
        cid = lax.axis_index("c")
        sid = lax.axis_index("s")
        wid = sid * 2 + cid
        lanes = lax.iota(jnp.int32, 16)
        ones = jnp.ones((16,), jnp.int32)
        izeros = jnp.zeros((16,), jnp.int32)
        fzeros = jnp.zeros((16,), jnp.float32)
        lane_hist = lanes * _NB
        lane_h2 = lanes * 16

        @plsc.parallel_loop(0, 16 * _NB // 16, unroll=4)
        def zero_hist(j):
            hist[pl.ds(j * 16, 16)] = izeros

        def row_step(i, acc):
            r = wid * rpw + i
            rn = r // c
            rc = r - rn * c
            pltpu.sync_copy(diff_hbm.at[rn, rc], row_buf)

            @plsc.parallel_loop(0, hh, unroll=2)
            def p1(s):
                for t in range(ngrp):
                    u = row_buf[s, pl.ds(t * 16, 16)]
                    blo = lax.shift_right_logical(u & 0xFFFF, 4)
                    bhi = lax.shift_right_logical(u, 20)
                    plsc.addupdate_scatter(hist, [lane_hist + blo], ones)
                    plsc.addupdate_scatter(hist, [lane_hist + bhi], ones)

            @plsc.parallel_loop(0, _NBC, unroll=2)
            def lred(cc):
                b0 = cc * 16
                v = hist[pl.ds(b0, 16)]
                hist[pl.ds(b0, 16)] = izeros
                for l in range(1, 16):
                    o = l * _NB + b0
                    v = v + hist[pl.ds(o, 16)]
                    hist[pl.ds(o, 16)] = izeros
                counts[pl.ds(b0, 16)] = v

            def walk(cc, st):
                carry, found, tb, need = st
                b0 = (_NBC - 1 - cc) * 16
                v = counts[pl.ds(b0, 16)]
                rev = lax.rev(v, (0,))
                cs = plsc.cumsum(rev) + carry
                reached = cs >= k
                nreach = jnp.sum(reached.astype(jnp.int32))
                p = jnp.min(jnp.where(reached, lanes, 16))
                cumbefore = cs - rev
                cb_p = jnp.sum(jnp.where(lanes == p, cumbefore, 0))
                upd = jnp.logical_and(nreach > 0, found == 0)
                tb = jnp.where(upd, b0 + 15 - p, tb)
                need = jnp.where(upd, k - cb_p, need)
                found = jnp.where(upd, 1, found)
                return (carry + jnp.sum(v), found, tb, need)
            _, _, tb, need = lax.fori_loop(
                0, _NBC, walk,
                (jnp.int32(0), jnp.int32(0), jnp.int32(0), jnp.int32(0)))

            for l in range(16):
                hist2[pl.ds(l * 16, 16)] = izeros

            @plsc.parallel_loop(0, hh, unroll=2,
                                carry=(fzeros, fzeros, fzeros, fzeros))
            def p2(s, accs):
                accs = list(accs)
                idx = 0
                for t in range(ngrp):
                    u = row_buf[s, pl.ds(t * 16, 16)]
                    for pat in (u & 0xFFFF, lax.shift_right_logical(u, 16)):
                        b = lax.shift_right_logical(pat, 4)
                        val = plsc.bitcast(lax.shift_left(pat, 16),
                                           jnp.float32)
                        accs[idx] = accs[idx] + jnp.where(b > tb, val, 0.0)
                        idx = (idx + 1) % 4
                        plsc.addupdate_scatter(
                            hist2, [lane_h2 + (pat & 15)], ones,
                            mask=b == tb)
                return tuple(accs)
            accv = p2[0] + p2[1] + p2[2] + p2[3]

            c2 = hist2[pl.ds(0, 16)]
            for l in range(1, 16):
                c2 = c2 + hist2[pl.ds(l * 16, 16)]
            rev2 = lax.rev(c2, (0,))
            cs2 = plsc.cumsum(rev2)
            take = jnp.clip(need - (cs2 - rev2), 0, rev2)
            pat = lax.shift_left(tb, 4) + (15 - lanes)
            vals = plsc.bitcast(lax.shift_left(pat, 16), jnp.float32)
            return acc + accv + take.astype(jnp.float32) * vals

        acc = lax.fori_loop(0, rpw, row_step, fzeros)
        for l in range(8):
            obuf[pl.ds(l * 16, 16)] = fzeros
        obuf[pl.ds(0, 16)] = acc
        pltpu.sync_copy(obuf, out_hbm.at[wid])

    return sc_topk


def kernel(input, target):
    n, c, h, w = input.shape
    k = int(h * w * _RATIO)
    packed = _diff(input, target)
    partials = _make_sc_topk(n, c, h // 2, w, k)(packed)
    out = _mean(partials, float(n * c) * float(k))
    return out.reshape(())
